# Initial kernel scaffold; baseline (speedup 1.0000x reference)
import jax
import jax.numpy as jnp
from jax.experimental import pallas as pl


def kernel(node_features, edge_index, edge_weight, target_node, w_n2l, bias_n2l, conv_W, conv_b, lin1_W, lin1_b, linout_W, linout_b):
    raise NotImplementedError("write your pallas kernel here")


# speedup vs baseline: 14.7552x; 14.7552x over previous
import functools

import jax
import jax.numpy as jnp
from jax import lax
from jax.experimental import pallas as pl
from jax.experimental.pallas import tpu as pltpu
from jax.experimental.pallas import tpu_sc as plsc

NC = 2
NS = 16
L = 16


def _rsqrt16(x):
    i = plsc.bitcast(x, jnp.int32)
    y = plsc.bitcast(jnp.int32(0x5F3759DF) - (i >> 1), jnp.float32)
    for _ in range(3):
        y = y * (1.5 - 0.5 * x * y * y)
    return jnp.where(x > 0.0, y, 0.0)


def _k1_body(x_ref, w_ref, b_ref, im_ref, ne_ref):
    im = jnp.dot(x_ref[...], w_ref[...], preferred_element_type=jnp.float32,
                 precision=lax.Precision.HIGHEST)
    im = im + b_ref[...]
    im_ref[...] = im
    ne_ref[...] = jnp.maximum(im, 0.0)


def _k3_body(p2_ref, dis_ref, im_ref, cw_ref, cb_ref, w1a_ref, w1b_ref,
             b1_ref, wo_ref, bo_ref, tgt_ref, out_ref):
    n = im_ref.shape[0]
    pool = (p2_ref[0] + p2_ref[1]) * dis_ref[...]
    nl = jnp.dot(pool, cw_ref[...], preferred_element_type=jnp.float32,
                 precision=lax.Precision.HIGHEST)
    ne2 = jnp.maximum(nl + cb_ref[...] + im_ref[...], 0.0)
    g = jnp.sum(ne2, axis=0, keepdims=True) * (1.0 / n)
    rows = lax.broadcasted_iota(jnp.int32, (n, 1), 0)
    tmask = rows == tgt_ref[0, 0]
    t = jnp.sum(jnp.where(tmask, ne2, 0.0), axis=0, keepdims=True)
    h1 = jnp.dot(ne2, w1a_ref[...], preferred_element_type=jnp.float32,
                 precision=lax.Precision.HIGHEST)
    gterm = jnp.dot(g, w1b_ref[...], preferred_element_type=jnp.float32,
                 precision=lax.Precision.HIGHEST)
    h1 = jnp.maximum(h1 + gterm + b1_ref[...], 0.0)
    raw = jnp.dot(h1, wo_ref[...], preferred_element_type=jnp.float32,
                 precision=lax.Precision.HIGHEST)
    raw = raw + bo_ref[...]
    out_ref[...] = jnp.dot(raw, t.T, preferred_element_type=jnp.float32,
                 precision=lax.Precision.HIGHEST)


def _k2_body(n, e, d, row_hbm, col_hbm, w_hbm, embed_hbm,
             pool2_hbm, dis_hbm,
             dis_v, rowa_v, wa_v, colf_v, wf_v, rowi_v,
             rows_v, zbuf_v, acc_v, tmp_v, pool_sh, parts_sh, dis_sh, sem):
    c = lax.axis_index("c")
    s = lax.axis_index("s")
    wid = s * NC + c

    npad = dis_v.shape[0]
    ea = e // NS
    eb = e // (NC * NS)
    bsz = 80
    nb = eb // bsz

    def zero_deg(i, _):
        dis_v[pl.ds(i * L, L)] = jnp.zeros((L,), jnp.float32)
        return 0
    lax.fori_loop(0, npad // L, zero_deg, 0)

    def zero_zbuf(i, _):
        for ch in range(d // L):
            zbuf_v[i, pl.ds(ch * L, L)] = jnp.zeros((L,), jnp.float32)
        return 0
    lax.fori_loop(0, zbuf_v.shape[0], zero_zbuf, 0)

    stripe = 640
    zr = zbuf_v.shape[0]
    @pl.when(s < NS - 1)
    def _():
        for q in range(stripe // zr):
            pltpu.sync_copy(zbuf_v,
                            pool_sh.at[pl.ds(s * stripe + q * zr, zr)])
    @pl.when(s == NS - 1)
    def _():
        for q in range((n - (NS - 1) * stripe) // zr):
            pltpu.sync_copy(zbuf_v,
                            pool_sh.at[pl.ds((NS - 1) * stripe + q * zr, zr)])

    basea = s * ea
    clen = rowa_v.shape[0]

    def deg_chunk(k, _):
        pltpu.sync_copy(row_hbm.at[pl.ds(basea + k * clen, clen)], rowa_v)
        pltpu.sync_copy(w_hbm.at[pl.ds(basea + k * clen, clen)], wa_v)

        lane = lax.broadcasted_iota(jnp.int32, (L,), 0)

        def deg_step(i, _):
            idx = rowa_v[pl.ds(i * L, L)]
            vals = wa_v[pl.ds(i * L, L)]
            for l in range(L):
                plsc.addupdate_scatter(dis_v, [idx], vals, mask=lane == l)
            return 0
        lax.fori_loop(0, clen // L, deg_step, 0)
        return 0
    lax.fori_loop(0, ea // clen, deg_chunk, 0)

    pltpu.sync_copy(dis_v, parts_sh.at[s])
    plsc.subcore_barrier()

    dstripe = npad // NS
    dbase = s * dstripe
    pltpu.sync_copy(parts_sh.at[0, pl.ds(dbase, dstripe)], acc_v)
    for t in range(1, NS):
        pltpu.sync_copy(parts_sh.at[t, pl.ds(dbase, dstripe)], tmp_v)
        def add_step(i, _):
            sl = pl.ds(i * L, L)
            acc_v[sl] = acc_v[sl] + tmp_v[sl]
            return 0
        lax.fori_loop(0, dstripe // L, add_step, 0)

    def rsq_step(i, _):
        sl = pl.ds(i * L, L)
        acc_v[sl] = _rsqrt16(acc_v[sl])
        return 0
    lax.fori_loop(0, dstripe // L, rsq_step, 0)
    pltpu.sync_copy(acc_v, dis_sh.at[pl.ds(dbase, dstripe)])
    plsc.subcore_barrier()
    pltpu.sync_copy(dis_sh, dis_v)

    baseb = wid * eb

    def spmm_step(j, _):
        pltpu.sync_copy(row_hbm.at[pl.ds(baseb + j * bsz, bsz)],
                        rowi_v.at[0])
        pltpu.sync_copy(col_hbm.at[pl.ds(baseb + j * bsz, bsz)], colf_v)
        pltpu.sync_copy(w_hbm.at[pl.ds(baseb + j * bsz, bsz)], wf_v)
        pltpu.async_copy(embed_hbm.at[colf_v], rows_v, sem).wait()
        for g in range(bsz // L):
            sl = pl.ds(g * L, L)
            cv = colf_v[sl]
            wv = wf_v[sl]
            dv = plsc.load_gather(dis_v, [cv])
            sv = wv * dv
            for l in range(L):
                r = g * L + l
                b = jnp.take(sv, jnp.full((L,), l, jnp.int32), mode='fill')
                for ch in range(d // L):
                    rows_v[r, pl.ds(ch * L, L)] = (
                        rows_v[r, pl.ds(ch * L, L)] * b)
        pltpu.sync_copy(rows_v, pool_sh.at[rowi_v.at[0]], add=True)
        return 0
    lax.fori_loop(0, nb, spmm_step, 0)

    plsc.subcore_barrier()

    @pl.when(s < NS - 1)
    def _():
        pltpu.sync_copy(pool_sh.at[pl.ds(s * stripe, stripe)],
                        pool2_hbm.at[c, pl.ds(s * stripe, stripe)])
    @pl.when(s == NS - 1)
    def _():
        rem = n - (NS - 1) * stripe
        pltpu.sync_copy(pool_sh.at[pl.ds((NS - 1) * stripe, rem)],
                        pool2_hbm.at[c, pl.ds((NS - 1) * stripe, rem)])
    @pl.when(jnp.logical_and(c == 0, s == 0))
    def _():
        pltpu.sync_copy(dis_v, dis_hbm)


def kernel(node_features, edge_index, edge_weight, target_node, w_n2l,
           bias_n2l, conv_W, conv_b, lin1_W, lin1_b, linout_W, linout_b):
    n, f_in = node_features.shape
    e = edge_weight.shape[0]
    d = w_n2l.shape[1]
    h = lin1_W.shape[1]

    row = edge_index[0]
    col = edge_index[1]

    im, ne = pl.pallas_call(
        _k1_body,
        out_shape=(
            jax.ShapeDtypeStruct((n, d), jnp.float32),
            jax.ShapeDtypeStruct((n, d), jnp.float32),
        ),
    )(node_features, w_n2l, bias_n2l.reshape(1, d))

    eb = e // (NC * NS)
    bsz = 80
    mesh = plsc.VectorSubcoreMesh(core_axis_name="c", subcore_axis_name="s",
                                  num_cores=NC, num_subcores=NS)
    npad = 10240
    k2 = pl.kernel(
        functools.partial(_k2_body, n, e, d),
        out_type=(
            jax.ShapeDtypeStruct((NC, n, d), jnp.float32),
            jax.ShapeDtypeStruct((npad,), jnp.float32),
        ),
        mesh=mesh,
        compiler_params=pltpu.CompilerParams(needs_layout_passes=False,
                                             use_tc_tiling_on_sc=False),
        scratch_types=[
            pltpu.VMEM((npad,), jnp.float32),
            pltpu.VMEM((4000,), jnp.int32),
            pltpu.VMEM((4000,), jnp.float32),
            pltpu.VMEM((bsz,), jnp.int32),
            pltpu.VMEM((bsz,), jnp.float32),
            pltpu.VMEM((1, bsz), jnp.int32),
            pltpu.VMEM((bsz, d), jnp.float32),
            pltpu.VMEM((80, d), jnp.float32),
            pltpu.VMEM((640,), jnp.float32),
            pltpu.VMEM((640,), jnp.float32),
            pltpu.VMEM_SHARED((n, d), jnp.float32),
            pltpu.VMEM_SHARED((NS, npad), jnp.float32),
            pltpu.VMEM_SHARED((npad,), jnp.float32),
            pltpu.SemaphoreType.DMA,
        ],
    )
    pool2, dispad = k2(row, col, edge_weight, ne)
    dis = dispad[:n]

    tgt = jnp.asarray(target_node, jnp.int32).reshape(1, 1)
    pred = pl.pallas_call(
        _k3_body,
        out_shape=jax.ShapeDtypeStruct((n, 1), jnp.float32),
    )(pool2, dis.reshape(n, 1), im, conv_W, conv_b.reshape(1, d),
      lin1_W[:d], lin1_W[d:], lin1_b.reshape(1, h), linout_W,
      linout_b.reshape(1, d), tgt)
    return pred


# speedup vs baseline: 27.9840x; 1.8966x over previous
import functools

import jax
import jax.numpy as jnp
from jax import lax
from jax.experimental import pallas as pl
from jax.experimental.pallas import tpu as pltpu
from jax.experimental.pallas import tpu_sc as plsc

NC = 2
NS = 16
L = 16


def _rsqrt16(x):
    i = plsc.bitcast(x, jnp.int32)
    y = plsc.bitcast(jnp.int32(0x5F3759DF) - (i >> 1), jnp.float32)
    for _ in range(3):
        y = y * (1.5 - 0.5 * x * y * y)
    return jnp.where(x > 0.0, y, 0.0)


def _k1_body(x_ref, w_ref, b_ref, im_ref, ne_ref):
    im = jnp.dot(x_ref[...], w_ref[...], preferred_element_type=jnp.float32,
                 precision=lax.Precision.HIGHEST)
    im = im + b_ref[...]
    im_ref[...] = im
    ne_ref[...] = jnp.maximum(im, 0.0)


def _k3_body(p2_ref, dis_ref, im_ref, cw_ref, cb_ref, w1a_ref, w1b_ref,
             b1_ref, wo_ref, bo_ref, tgt_ref, out_ref):
    n = im_ref.shape[0]
    pool = (p2_ref[0] + p2_ref[1]) * dis_ref[...]
    nl = jnp.dot(pool, cw_ref[...], preferred_element_type=jnp.float32,
                 precision=lax.Precision.HIGHEST)
    ne2 = jnp.maximum(nl + cb_ref[...] + im_ref[...], 0.0)
    g = jnp.sum(ne2, axis=0, keepdims=True) * (1.0 / n)
    rows = lax.broadcasted_iota(jnp.int32, (n, 1), 0)
    tmask = rows == tgt_ref[0, 0]
    t = jnp.sum(jnp.where(tmask, ne2, 0.0), axis=0, keepdims=True)
    h1 = jnp.dot(ne2, w1a_ref[...], preferred_element_type=jnp.float32,
                 precision=lax.Precision.HIGHEST)
    gterm = jnp.dot(g, w1b_ref[...], preferred_element_type=jnp.float32,
                 precision=lax.Precision.HIGHEST)
    h1 = jnp.maximum(h1 + gterm + b1_ref[...], 0.0)
    raw = jnp.dot(h1, wo_ref[...], preferred_element_type=jnp.float32,
                 precision=lax.Precision.HIGHEST)
    raw = raw + bo_ref[...]
    out_ref[...] = jnp.dot(raw, t.T, preferred_element_type=jnp.float32,
                 precision=lax.Precision.HIGHEST)


def _k2_body(n, e, d, row_hbm, col_hbm, w_hbm, embed_hbm,
             pool2_hbm, dis_hbm,
             dis_v, rowa_v, wa_v, colf_v, wf_v, rowi_v,
             rowsa_v, rowsb_v, zbuf_v, acc_v, tmp_v, pool_sh, parts_sh,
             dis_sh, sem, gsa, gsb, rsa, rsb):
    c = lax.axis_index("c")
    s = lax.axis_index("s")
    wid = s * NC + c

    npad = dis_v.shape[0]
    ea = e // NS
    eb = e // (NC * NS)
    bsz = 80
    nb = eb // bsz
    baseb = wid * eb

    stg_col = pltpu.async_copy(col_hbm.at[pl.ds(baseb, eb)], colf_v, sem)
    stg_w = pltpu.async_copy(w_hbm.at[pl.ds(baseb, eb)], wf_v, sem)

    def zero_deg(i, _):
        dis_v[pl.ds(i * L, L)] = jnp.zeros((L,), jnp.float32)
        return 0
    lax.fori_loop(0, npad // L, zero_deg, 0)

    def zero_zbuf(i, _):
        for ch in range(d // L):
            zbuf_v[i, pl.ds(ch * L, L)] = jnp.zeros((L,), jnp.float32)
        return 0
    lax.fori_loop(0, zbuf_v.shape[0], zero_zbuf, 0)

    stripe = 640
    zr = zbuf_v.shape[0]
    @pl.when(s < NS - 1)
    def _():
        for q in range(stripe // zr):
            pltpu.sync_copy(zbuf_v,
                            pool_sh.at[pl.ds(s * stripe + q * zr, zr)])
    @pl.when(s == NS - 1)
    def _():
        for q in range((n - (NS - 1) * stripe) // zr):
            pltpu.sync_copy(zbuf_v,
                            pool_sh.at[pl.ds((NS - 1) * stripe + q * zr, zr)])

    basea = s * ea
    clen = rowa_v.shape[0]

    def deg_chunk(k, _):
        pltpu.sync_copy(row_hbm.at[pl.ds(basea + k * clen, clen)], rowa_v)
        pltpu.sync_copy(w_hbm.at[pl.ds(basea + k * clen, clen)], wa_v)

        lane = lax.broadcasted_iota(jnp.int32, (L,), 0)

        def deg_step(i, _):
            idx = rowa_v[pl.ds(i * L, L)]
            vals = wa_v[pl.ds(i * L, L)]
            for l in range(L):
                plsc.addupdate_scatter(dis_v, [idx], vals, mask=lane == l)
            return 0
        lax.fori_loop(0, clen // L, deg_step, 0)
        return 0
    lax.fori_loop(0, ea // clen, deg_chunk, 0)

    pltpu.sync_copy(dis_v, parts_sh.at[s])
    plsc.subcore_barrier()

    dstripe = npad // NS
    dbase = s * dstripe
    pltpu.sync_copy(parts_sh.at[0, pl.ds(dbase, dstripe)], acc_v)
    for t in range(1, NS):
        pltpu.sync_copy(parts_sh.at[t, pl.ds(dbase, dstripe)], tmp_v)
        def add_step(i, _):
            sl = pl.ds(i * L, L)
            acc_v[sl] = acc_v[sl] + tmp_v[sl]
            return 0
        lax.fori_loop(0, dstripe // L, add_step, 0)

    def rsq_step(i, _):
        sl = pl.ds(i * L, L)
        acc_v[sl] = _rsqrt16(acc_v[sl])
        return 0
    lax.fori_loop(0, dstripe // L, rsq_step, 0)
    pltpu.sync_copy(acc_v, dis_sh.at[pl.ds(dbase, dstripe)])
    plsc.subcore_barrier()
    pltpu.sync_copy(dis_sh, dis_v)

    stg_col.wait()
    stg_w.wait()

    def g_idx(j):
        return embed_hbm.at[colf_v.at[pl.ds(j * bsz, bsz)]]

    def r_src(j):
        return row_hbm.at[pl.ds(baseb + j * bsz, bsz)]

    def issue(j, buf, rslot, gsem, rsem):
        pltpu.async_copy(g_idx(j), buf, gsem)
        pltpu.async_copy(r_src(j), rowi_v.at[rslot], rsem)

    def wait_in(j, buf, rslot, gsem, rsem):
        pltpu.make_async_copy(g_idx(j), buf, gsem).wait()
        pltpu.make_async_copy(r_src(j), rowi_v.at[rslot], rsem).wait()

    def scale(j, buf):
        for g in range(bsz // L):
            sl = pl.ds(j * bsz + g * L, L)
            cv = colf_v[sl]
            wv = wf_v[sl]
            dv = plsc.load_gather(dis_v, [cv])
            sv = wv * dv
            for l in range(L):
                r = g * L + l
                b = jnp.take(sv, jnp.full((L,), l, jnp.int32), mode='fill')
                for ch in range(d // L):
                    buf[r, pl.ds(ch * L, L)] = buf[r, pl.ds(ch * L, L)] * b

    def scatter(buf, rslot):
        pltpu.sync_copy(buf, pool_sh.at[rowi_v.at[rslot]], add=True)

    issue(0, rowsa_v, 0, gsa, rsa)

    def spmm_pair(j2, _):
        j = 2 * j2
        wait_in(j, rowsa_v, 0, gsa, rsa)
        issue(j + 1, rowsb_v, 1, gsb, rsb)
        scale(j, rowsa_v)
        scatter(rowsa_v, 0)
        wait_in(j + 1, rowsb_v, 1, gsb, rsb)
        @pl.when(j + 2 < nb)
        def _():
            issue(j + 2, rowsa_v, 0, gsa, rsa)
        scale(j + 1, rowsb_v)
        scatter(rowsb_v, 1)
        return 0
    lax.fori_loop(0, nb // 2, spmm_pair, 0)
    jl = nb - 1
    wait_in(jl, rowsa_v, 0, gsa, rsa)
    scale(jl, rowsa_v)
    scatter(rowsa_v, 0)

    plsc.subcore_barrier()

    @pl.when(s < NS - 1)
    def _():
        pltpu.sync_copy(pool_sh.at[pl.ds(s * stripe, stripe)],
                        pool2_hbm.at[c, pl.ds(s * stripe, stripe)])
    @pl.when(s == NS - 1)
    def _():
        rem = n - (NS - 1) * stripe
        pltpu.sync_copy(pool_sh.at[pl.ds((NS - 1) * stripe, rem)],
                        pool2_hbm.at[c, pl.ds((NS - 1) * stripe, rem)])
    @pl.when(jnp.logical_and(c == 0, s == 0))
    def _():
        pltpu.sync_copy(dis_v, dis_hbm)


def kernel(node_features, edge_index, edge_weight, target_node, w_n2l,
           bias_n2l, conv_W, conv_b, lin1_W, lin1_b, linout_W, linout_b):
    n, f_in = node_features.shape
    e = edge_weight.shape[0]
    d = w_n2l.shape[1]
    h = lin1_W.shape[1]

    row = edge_index[0]
    col = edge_index[1]

    im, ne = pl.pallas_call(
        _k1_body,
        out_shape=(
            jax.ShapeDtypeStruct((n, d), jnp.float32),
            jax.ShapeDtypeStruct((n, d), jnp.float32),
        ),
    )(node_features, w_n2l, bias_n2l.reshape(1, d))

    eb = e // (NC * NS)
    bsz = 80
    mesh = plsc.VectorSubcoreMesh(core_axis_name="c", subcore_axis_name="s",
                                  num_cores=NC, num_subcores=NS)
    npad = 10240
    k2 = pl.kernel(
        functools.partial(_k2_body, n, e, d),
        out_type=(
            jax.ShapeDtypeStruct((NC, n, d), jnp.float32),
            jax.ShapeDtypeStruct((npad,), jnp.float32),
        ),
        mesh=mesh,
        compiler_params=pltpu.CompilerParams(needs_layout_passes=False,
                                             use_tc_tiling_on_sc=False),
        scratch_types=[
            pltpu.VMEM((npad,), jnp.float32),
            pltpu.VMEM((4000,), jnp.int32),
            pltpu.VMEM((4000,), jnp.float32),
            pltpu.VMEM((eb,), jnp.int32),
            pltpu.VMEM((eb,), jnp.float32),
            pltpu.VMEM((2, bsz), jnp.int32),
            pltpu.VMEM((bsz, d), jnp.float32),
            pltpu.VMEM((bsz, d), jnp.float32),
            pltpu.VMEM((80, d), jnp.float32),
            pltpu.VMEM((640,), jnp.float32),
            pltpu.VMEM((640,), jnp.float32),
            pltpu.VMEM_SHARED((n, d), jnp.float32),
            pltpu.VMEM_SHARED((NS, npad), jnp.float32),
            pltpu.VMEM_SHARED((npad,), jnp.float32),
            pltpu.SemaphoreType.DMA,
            pltpu.SemaphoreType.DMA,
            pltpu.SemaphoreType.DMA,
            pltpu.SemaphoreType.DMA,
            pltpu.SemaphoreType.DMA,
        ],
    )
    pool2, dispad = k2(row, col, edge_weight, ne)
    dis = dispad[:n]

    tgt = jnp.asarray(target_node, jnp.int32).reshape(1, 1)
    pred = pl.pallas_call(
        _k3_body,
        out_shape=jax.ShapeDtypeStruct((n, 1), jnp.float32),
    )(pool2, dis.reshape(n, 1), im, conv_W, conv_b.reshape(1, d),
      lin1_W[:d], lin1_W[d:], lin1_b.reshape(1, h), linout_W,
      linout_b.reshape(1, d), tgt)
    return pred


# speedup vs baseline: 30.8034x; 1.1008x over previous
import functools

import jax
import jax.numpy as jnp
from jax import lax
from jax.experimental import pallas as pl
from jax.experimental.pallas import tpu as pltpu
from jax.experimental.pallas import tpu_sc as plsc

NC = 2
NS = 16
L = 16


def _rsqrt16(x):
    i = plsc.bitcast(x, jnp.int32)
    y = plsc.bitcast(jnp.int32(0x5F3759DF) - (i >> 1), jnp.float32)
    for _ in range(3):
        y = y * (1.5 - 0.5 * x * y * y)
    return jnp.where(x > 0.0, y, 0.0)


def _k1_body(x_ref, w_ref, b_ref, im_ref, ne_ref):
    im = jnp.dot(x_ref[...], w_ref[...], preferred_element_type=jnp.float32,
                 precision=lax.Precision.HIGHEST)
    im = im + b_ref[...]
    im_ref[...] = im
    ne_ref[...] = jnp.maximum(im, 0.0)


def _k3_body(p2_ref, dis_ref, im_ref, cw_ref, cb_ref, w1a_ref, w1b_ref,
             b1_ref, wo_ref, bo_ref, tgt_ref, out_ref):
    n = im_ref.shape[0]
    pool = (p2_ref[0] + p2_ref[1]) * dis_ref[...]
    nl = jnp.dot(pool, cw_ref[...], preferred_element_type=jnp.float32,
                 precision=lax.Precision.HIGHEST)
    ne2 = jnp.maximum(nl + cb_ref[...] + im_ref[...], 0.0)
    g = jnp.sum(ne2, axis=0, keepdims=True) * (1.0 / n)
    rows = lax.broadcasted_iota(jnp.int32, (n, 1), 0)
    tmask = rows == tgt_ref[0, 0]
    t = jnp.sum(jnp.where(tmask, ne2, 0.0), axis=0, keepdims=True)
    h1 = jnp.dot(ne2, w1a_ref[...], preferred_element_type=jnp.float32,
                 precision=lax.Precision.HIGHEST)
    gterm = jnp.dot(g, w1b_ref[...], preferred_element_type=jnp.float32,
                 precision=lax.Precision.HIGHEST)
    h1 = jnp.maximum(h1 + gterm + b1_ref[...], 0.0)
    raw = jnp.dot(h1, wo_ref[...], preferred_element_type=jnp.float32,
                 precision=lax.Precision.HIGHEST)
    raw = raw + bo_ref[...]
    out_ref[...] = jnp.dot(raw, t.T, preferred_element_type=jnp.float32,
                 precision=lax.Precision.HIGHEST)


def _k2_body(n, e, d, row_hbm, col_hbm, w_hbm, embed_hbm,
             pool2_hbm, dis_hbm,
             dis_v, rowa_v, wa_v, colf_v, wf_v, rowi_v, rowit_v,
             rowsa_v, rowsb_v, rowsc_v, zbuf_v, acc_v, tmp_v, pool_sh,
             parts_sh, dis_sh, sem, gs0, gs1, gs2, rs0, rs1, rs2,
             ss0, ss1, ss2):
    c = lax.axis_index("c")
    s = lax.axis_index("s")
    wid = s * NC + c

    npad = dis_v.shape[0]
    ea = e // NS
    eb = e // (NC * NS)
    bsz = 128
    nb = eb // bsz
    tail = eb - nb * bsz
    baseb = wid * eb
    bufs = (rowsa_v, rowsb_v, rowsc_v)
    gsems = (gs0, gs1, gs2)
    rsems = (rs0, rs1, rs2)
    ssems = (ss0, ss1, ss2)

    stg_col = pltpu.async_copy(col_hbm.at[pl.ds(baseb, eb)], colf_v, sem)
    stg_w = pltpu.async_copy(w_hbm.at[pl.ds(baseb, eb)], wf_v, sem)

    def zero_deg(i, _):
        dis_v[pl.ds(i * L, L)] = jnp.zeros((L,), jnp.float32)
        return 0
    lax.fori_loop(0, npad // L, zero_deg, 0)

    def zero_zbuf(i, _):
        for ch in range(d // L):
            zbuf_v[i, pl.ds(ch * L, L)] = jnp.zeros((L,), jnp.float32)
        return 0
    lax.fori_loop(0, zbuf_v.shape[0], zero_zbuf, 0)

    stripe = 640
    zr = zbuf_v.shape[0]
    @pl.when(s < NS - 1)
    def _():
        for q in range(stripe // zr):
            pltpu.sync_copy(zbuf_v,
                            pool_sh.at[pl.ds(s * stripe + q * zr, zr)])
    @pl.when(s == NS - 1)
    def _():
        for q in range((n - (NS - 1) * stripe) // zr):
            pltpu.sync_copy(zbuf_v,
                            pool_sh.at[pl.ds((NS - 1) * stripe + q * zr, zr)])

    basea = s * ea
    clen = rowa_v.shape[0]

    def deg_chunk(k, _):
        pltpu.sync_copy(row_hbm.at[pl.ds(basea + k * clen, clen)], rowa_v)
        pltpu.sync_copy(w_hbm.at[pl.ds(basea + k * clen, clen)], wa_v)

        lane = lax.broadcasted_iota(jnp.int32, (L,), 0)

        def deg_step(i, _):
            idx = rowa_v[pl.ds(i * L, L)]
            vals = wa_v[pl.ds(i * L, L)]
            for l in range(L):
                plsc.addupdate_scatter(dis_v, [idx], vals, mask=lane == l)
            return 0
        lax.fori_loop(0, clen // L, deg_step, 0)
        return 0
    lax.fori_loop(0, ea // clen, deg_chunk, 0)

    pltpu.sync_copy(dis_v, parts_sh.at[s])
    plsc.subcore_barrier()

    dstripe = npad // NS
    dbase = s * dstripe
    pltpu.sync_copy(parts_sh.at[0, pl.ds(dbase, dstripe)], acc_v)
    for t in range(1, NS):
        pltpu.sync_copy(parts_sh.at[t, pl.ds(dbase, dstripe)], tmp_v)
        def add_step(i, _):
            sl = pl.ds(i * L, L)
            acc_v[sl] = acc_v[sl] + tmp_v[sl]
            return 0
        lax.fori_loop(0, dstripe // L, add_step, 0)

    def rsq_step(i, _):
        sl = pl.ds(i * L, L)
        acc_v[sl] = _rsqrt16(acc_v[sl])
        return 0
    lax.fori_loop(0, dstripe // L, rsq_step, 0)
    pltpu.sync_copy(acc_v, dis_sh.at[pl.ds(dbase, dstripe)])
    plsc.subcore_barrier()
    pltpu.sync_copy(dis_sh, dis_v)

    stg_col.wait()
    stg_w.wait()

    def g_idx(j):
        return embed_hbm.at[colf_v.at[pl.ds(j * bsz, bsz)]]

    def r_src(j):
        return row_hbm.at[pl.ds(baseb + j * bsz, bsz)]

    def issue_in(j, k):
        pltpu.async_copy(g_idx(j), bufs[k], gsems[k])
        pltpu.async_copy(r_src(j), rowi_v.at[k], rsems[k])

    def wait_in(j, k):
        pltpu.make_async_copy(g_idx(j), bufs[k], gsems[k]).wait()
        pltpu.make_async_copy(r_src(j), rowi_v.at[k], rsems[k]).wait()

    def scale(j, buf, m):
        for g in range(m // L):
            sl = pl.ds(j * bsz + g * L, L)
            cv = colf_v[sl]
            wv = wf_v[sl]
            dv = plsc.load_gather(dis_v, [cv])
            sv = wv * dv
            for l in range(L):
                r = g * L + l
                b = jnp.take(sv, jnp.full((L,), l, jnp.int32), mode='fill')
                for ch in range(d // L):
                    buf[r, pl.ds(ch * L, L)] = buf[r, pl.ds(ch * L, L)] * b

    def issue_s(k):
        pltpu.async_copy(bufs[k], pool_sh.at[rowi_v.at[k]], ssems[k],
                         add=True)

    def wait_s(k):
        pltpu.make_async_copy(bufs[k], pool_sh.at[rowi_v.at[k]],
                              ssems[k]).wait()

    def step(j, k):
        wait_in(j, k)
        scale(j, bufs[k], bsz)
        kn = (k + 2) % 3
        @pl.when(j + 2 < nb)
        def _():
            @pl.when(j >= 1)
            def _():
                wait_s(kn)
            issue_in(j + 2, kn)
        issue_s(k)

    issue_in(0, 0)
    issue_in(1, 1)

    def spmm_triple(t, _):
        j = 3 * t
        step(j, 0)
        step(j + 1, 1)
        step(j + 2, 2)
        return 0
    lax.fori_loop(0, nb // 3, spmm_triple, 0)
    for k in range(3):
        wait_s(k)
    if tail:
        jt = nb * bsz
        pltpu.async_copy(
            embed_hbm.at[colf_v.at[pl.ds(jt, tail)]],
            rowsa_v.at[pl.ds(0, tail)], gs0).wait()
        pltpu.async_copy(row_hbm.at[pl.ds(baseb + jt, tail)],
                         rowit_v.at[0], rs0).wait()
        cv = colf_v[pl.ds(jt, L)]
        wv = wf_v[pl.ds(jt, L)]
        sv = wv * plsc.load_gather(dis_v, [cv])
        for l in range(L):
            b = jnp.take(sv, jnp.full((L,), l, jnp.int32), mode='fill')
            for ch in range(d // L):
                rowsa_v[l, pl.ds(ch * L, L)] = (
                    rowsa_v[l, pl.ds(ch * L, L)] * b)
        pltpu.sync_copy(rowsa_v.at[pl.ds(0, tail)],
                        pool_sh.at[rowit_v.at[0]], add=True)

    plsc.subcore_barrier()

    @pl.when(s < NS - 1)
    def _():
        pltpu.sync_copy(pool_sh.at[pl.ds(s * stripe, stripe)],
                        pool2_hbm.at[c, pl.ds(s * stripe, stripe)])
    @pl.when(s == NS - 1)
    def _():
        rem = n - (NS - 1) * stripe
        pltpu.sync_copy(pool_sh.at[pl.ds((NS - 1) * stripe, rem)],
                        pool2_hbm.at[c, pl.ds((NS - 1) * stripe, rem)])
    @pl.when(jnp.logical_and(c == 0, s == 0))
    def _():
        pltpu.sync_copy(dis_v, dis_hbm)


def kernel(node_features, edge_index, edge_weight, target_node, w_n2l,
           bias_n2l, conv_W, conv_b, lin1_W, lin1_b, linout_W, linout_b):
    n, f_in = node_features.shape
    e = edge_weight.shape[0]
    d = w_n2l.shape[1]
    h = lin1_W.shape[1]

    row = edge_index[0]
    col = edge_index[1]

    im, ne = pl.pallas_call(
        _k1_body,
        out_shape=(
            jax.ShapeDtypeStruct((n, d), jnp.float32),
            jax.ShapeDtypeStruct((n, d), jnp.float32),
        ),
    )(node_features, w_n2l, bias_n2l.reshape(1, d))

    eb = e // (NC * NS)
    bsz = 128
    mesh = plsc.VectorSubcoreMesh(core_axis_name="c", subcore_axis_name="s",
                                  num_cores=NC, num_subcores=NS)
    npad = 10240
    k2 = pl.kernel(
        functools.partial(_k2_body, n, e, d),
        out_type=(
            jax.ShapeDtypeStruct((NC, n, d), jnp.float32),
            jax.ShapeDtypeStruct((npad,), jnp.float32),
        ),
        mesh=mesh,
        compiler_params=pltpu.CompilerParams(needs_layout_passes=False,
                                             use_tc_tiling_on_sc=False),
        scratch_types=[
            pltpu.VMEM((npad,), jnp.float32),
            pltpu.VMEM((4000,), jnp.int32),
            pltpu.VMEM((4000,), jnp.float32),
            pltpu.VMEM((eb,), jnp.int32),
            pltpu.VMEM((eb,), jnp.float32),
            pltpu.VMEM((3, bsz), jnp.int32),
            pltpu.VMEM((1, 16), jnp.int32),
            pltpu.VMEM((bsz, d), jnp.float32),
            pltpu.VMEM((bsz, d), jnp.float32),
            pltpu.VMEM((bsz, d), jnp.float32),
            pltpu.VMEM((80, d), jnp.float32),
            pltpu.VMEM((640,), jnp.float32),
            pltpu.VMEM((640,), jnp.float32),
            pltpu.VMEM_SHARED((n, d), jnp.float32),
            pltpu.VMEM_SHARED((NS, npad), jnp.float32),
            pltpu.VMEM_SHARED((npad,), jnp.float32),
        ] + [pltpu.SemaphoreType.DMA] * 10,
    )
    pool2, dispad = k2(row, col, edge_weight, ne)
    dis = dispad[:n]

    tgt = jnp.asarray(target_node, jnp.int32).reshape(1, 1)
    pred = pl.pallas_call(
        _k3_body,
        out_shape=jax.ShapeDtypeStruct((n, 1), jnp.float32),
    )(pool2, dis.reshape(n, 1), im, conv_W, conv_b.reshape(1, d),
      lin1_W[:d], lin1_W[d:], lin1_b.reshape(1, h), linout_W,
      linout_b.reshape(1, d), tgt)
    return pred
